# baseline (speedup 1.0000x reference)
import functools
import math

import jax
import jax.numpy as jnp
from jax import lax
from jax.experimental import pallas as pl
from jax.experimental.pallas import tpu as pltpu
from jax.experimental.pallas import tpu_sc as plsc

L = 16


@functools.lru_cache(maxsize=None)
def _make_sc_gather(B, V, D, C):
    info = plsc.get_sparse_core_info()
    NC, NS = info.num_cores, info.num_subcores
    NW = NC * NS
    assert B % NW == 0
    b_per_w = B // NW
    assert b_per_w % C == 0
    n_chunks = b_per_w // C
    scale = jnp.float32(math.sqrt(D))

    mesh = plsc.VectorSubcoreMesh(core_axis_name="c", subcore_axis_name="s")

    @functools.partial(
        pl.kernel,
        mesh=mesh,
        out_type=jax.ShapeDtypeStruct((B, D), jnp.float32),
        scratch_types=[
            pltpu.VMEM((b_per_w,), jnp.int32),
            pltpu.VMEM((C, D), jnp.float32),
            pltpu.SemaphoreType.DMA,
        ],
    )
    def k(idx_hbm, table_hbm, out_hbm, idx_v, rows_v, sem):
        wid = lax.axis_index("s") * NC + lax.axis_index("c")
        base = wid * b_per_w
        pltpu.sync_copy(idx_hbm.at[pl.ds(base, b_per_w)], idx_v)

        def chunk_body(g, _):
            pltpu.async_copy(
                table_hbm.at[idx_v.at[pl.ds(g * C, C)]], rows_v, sem
            ).wait()

            def row_body(r, _):
                def vec_body(j, _):
                    sl = pl.ds(j * L, L)
                    rows_v[r, sl] = rows_v[r, sl] * scale
                    return 0

                return lax.fori_loop(0, D // L, vec_body, 0, unroll=4)

            lax.fori_loop(0, C, row_body, 0)
            pltpu.sync_copy(rows_v, out_hbm.at[pl.ds(base + g * C, C)])
            return 0

        lax.fori_loop(0, n_chunks, chunk_body, 0)

    return k


def kernel(token_ids, W):
    B = token_ids.shape[0] * token_ids.shape[1]
    V, D = W.shape
    idx = token_ids.reshape(B).astype(jnp.int32)
    out = _make_sc_gather(B, V, D, 64)(idx, W)
    return out.reshape(token_ids.shape[0], token_ids.shape[1], D)


# speedup vs baseline: 2.8812x; 2.8812x over previous
import functools
import math

import jax
import jax.numpy as jnp
from jax import lax
from jax.experimental import pallas as pl
from jax.experimental.pallas import tpu as pltpu
from jax.experimental.pallas import tpu_sc as plsc

L = 16


@functools.lru_cache(maxsize=None)
def _make_sc_gather(B, V, D, C):
    info = plsc.get_sparse_core_info()
    NC, NS = info.num_cores, info.num_subcores
    NW = NC * NS
    assert B % NW == 0
    b_per_w = B // NW
    assert b_per_w % (2 * C) == 0
    n_pairs = b_per_w // (2 * C)
    scale = jnp.float32(math.sqrt(D))

    mesh = plsc.VectorSubcoreMesh(core_axis_name="c", subcore_axis_name="s")

    @functools.partial(
        pl.kernel,
        mesh=mesh,
        out_type=jax.ShapeDtypeStruct((B, D), jnp.float32),
        scratch_types=[
            pltpu.VMEM((b_per_w,), jnp.int32),
            pltpu.VMEM((C, D), jnp.float32),
            pltpu.VMEM((C, D), jnp.float32),
            pltpu.SemaphoreType.DMA,
            pltpu.SemaphoreType.DMA,
            pltpu.SemaphoreType.DMA,
            pltpu.SemaphoreType.DMA,
        ],
    )
    def k(idx_hbm, table_hbm, out_hbm, idx_v, buf0, buf1, sg0, sg1, ss0, ss1):
        wid = lax.axis_index("s") * NC + lax.axis_index("c")
        base = wid * b_per_w
        pltpu.sync_copy(idx_hbm.at[pl.ds(base, b_per_w)], idx_v)

        def start_gather(g, buf, sem):
            pltpu.async_copy(table_hbm.at[idx_v.at[pl.ds(g * C, C)]], buf, sem)

        def wait_gather(g, buf, sem):
            pltpu.make_async_copy(
                table_hbm.at[idx_v.at[pl.ds(g * C, C)]], buf, sem
            ).wait()

        def start_scatter(g, buf, sem):
            pltpu.async_copy(buf, out_hbm.at[pl.ds(base + g * C, C)], sem)

        def wait_scatter(g, buf, sem):
            pltpu.make_async_copy(
                buf, out_hbm.at[pl.ds(base + g * C, C)], sem
            ).wait()

        def scale_buf(buf):
            def row_body(r, _):
                for j in range(D // L):
                    sl = pl.ds(j * L, L)
                    buf[r, sl] = buf[r, sl] * scale
                return 0

            lax.fori_loop(0, C, row_body, 0)

        start_gather(0, buf0, sg0)
        start_gather(1, buf1, sg1)

        def pair_body(i, _):
            g0 = 2 * i
            g1 = g0 + 1
            wait_gather(g0, buf0, sg0)
            scale_buf(buf0)
            start_scatter(g0, buf0, ss0)
            wait_gather(g1, buf1, sg1)
            scale_buf(buf1)
            start_scatter(g1, buf1, ss1)

            @pl.when(i < n_pairs - 1)
            def _():
                wait_scatter(g0, buf0, ss0)
                start_gather(g0 + 2, buf0, sg0)
                wait_scatter(g1, buf1, ss1)
                start_gather(g1 + 2, buf1, sg1)

            return 0

        lax.fori_loop(0, n_pairs, pair_body, 0)
        wait_scatter(2 * n_pairs - 2, buf0, ss0)
        wait_scatter(2 * n_pairs - 1, buf1, ss1)

    return k


def kernel(token_ids, W):
    B = token_ids.shape[0] * token_ids.shape[1]
    V, D = W.shape
    idx = token_ids.reshape(B).astype(jnp.int32)
    out = _make_sc_gather(B, V, D, 32)(idx, W)
    return out.reshape(token_ids.shape[0], token_ids.shape[1], D)


# speedup vs baseline: 3.1247x; 1.0845x over previous
import functools
import math

import jax
import jax.numpy as jnp
from jax import lax
from jax.experimental import pallas as pl
from jax.experimental.pallas import tpu as pltpu
from jax.experimental.pallas import tpu_sc as plsc

L = 16


@functools.lru_cache(maxsize=None)
def _make_sc_gather(B, V, D, C):
    info = plsc.get_sparse_core_info()
    NC, NS = info.num_cores, info.num_subcores
    NW = NC * NS
    assert B % NW == 0
    b_per_w = B // NW
    assert b_per_w % C == 0
    n_chunks = b_per_w // C
    NBUF = 3
    scale = jnp.float32(math.sqrt(D))

    mesh = plsc.VectorSubcoreMesh(core_axis_name="c", subcore_axis_name="s")

    @functools.partial(
        pl.kernel,
        mesh=mesh,
        out_type=jax.ShapeDtypeStruct((B, D), jnp.float32),
        scratch_types=[
            pltpu.VMEM((b_per_w,), jnp.int32),
        ]
        + [pltpu.VMEM((C, D), jnp.float32)] * NBUF
        + [pltpu.SemaphoreType.DMA] * (2 * NBUF),
    )
    def k(idx_hbm, table_hbm, out_hbm, idx_v, *scratch):
        bufs = scratch[:NBUF]
        gsems = scratch[NBUF : 2 * NBUF]
        ssems = scratch[2 * NBUF : 3 * NBUF]
        wid = lax.axis_index("s") * NC + lax.axis_index("c")
        base = wid * b_per_w
        pltpu.sync_copy(idx_hbm.at[pl.ds(base, b_per_w)], idx_v)

        def start_gather(g):
            b = g % NBUF
            pltpu.async_copy(
                table_hbm.at[idx_v.at[pl.ds(g * C, C)]], bufs[b], gsems[b]
            )

        def wait_gather(g):
            b = g % NBUF
            pltpu.make_async_copy(
                table_hbm.at[idx_v.at[pl.ds(g * C, C)]], bufs[b], gsems[b]
            ).wait()

        def start_scatter(g):
            b = g % NBUF
            pltpu.async_copy(
                bufs[b], out_hbm.at[pl.ds(base + g * C, C)], ssems[b]
            )

        def wait_scatter(g):
            b = g % NBUF
            pltpu.make_async_copy(
                bufs[b], out_hbm.at[pl.ds(base + g * C, C)], ssems[b]
            ).wait()

        def scale_buf(buf):
            def row_body(r, _):
                for j in range(D // L):
                    sl = pl.ds(j * L, L)
                    buf[r, sl] = buf[r, sl] * scale
                return 0

            lax.fori_loop(0, C, row_body, 0)

        for g in range(NBUF):
            start_gather(g)
        for g in range(n_chunks):
            wait_gather(g)
            nxt = g + 1
            if NBUF - 1 <= g < n_chunks - 1:
                wait_scatter(nxt - NBUF)
                start_gather(nxt)
            scale_buf(bufs[g % NBUF])
            start_scatter(g)
        for g in range(n_chunks - NBUF, n_chunks):
            wait_scatter(g)

    return k


def kernel(token_ids, W):
    B = token_ids.shape[0] * token_ids.shape[1]
    V, D = W.shape
    idx = token_ids.reshape(B).astype(jnp.int32)
    out = _make_sc_gather(B, V, D, 32)(idx, W)
    return out.reshape(token_ids.shape[0], token_ids.shape[1], D)
